# baseline (speedup 1.0000x reference)
import functools

import jax
import jax.numpy as jnp
from jax import lax
from jax.experimental import pallas as pl
from jax.experimental.pallas import tpu as pltpu
from jax.experimental.pallas import tpu_sc as plsc

_NEG = float("-inf")


def _affine(x, w_ref, b_ref):
    return jnp.dot(x, w_ref[...], preferred_element_type=jnp.float32) + b_ref[...]


def _seg_cummax(m, ids, rows):
    ncol = m.shape[1]
    d = 1
    while d < rows:
        pad_m = jnp.full((d, ncol), _NEG, jnp.float32)
        pad_i = jnp.full((d, 1), -1, ids.dtype)
        sm = jnp.concatenate([pad_m, m[: rows - d]], axis=0)
        si = jnp.concatenate([pad_i, ids[: rows - d]], axis=0)
        m = jnp.where(ids == si, jnp.maximum(m, sm), m)
        d *= 2
    return m


def _carry_fold(hp, ids, carry_ref, cid_ref, rows, ncol):
    cvec = carry_ref[0:1, 0:ncol]
    hp = jnp.where(ids == cid_ref[0], jnp.maximum(hp, cvec), hp)
    m = _seg_cummax(hp, ids, rows)
    cid_ref[0] = ids[rows - 1, 0]
    carry_ref[0:1, 0:ncol] = m[rows - 1 : rows, :]
    return m


def _stage_call(body, nblk, blk, ncol, inputs, in_specs):
    grid = (nblk + 1,)
    out = pl.pallas_call(
        body,
        grid=grid,
        in_specs=in_specs,
        out_specs=pl.BlockSpec((blk, ncol), lambda i: (i, 0)),
        out_shape=jax.ShapeDtypeStruct(((nblk + 1) * blk, ncol), jnp.float32),
        scratch_shapes=[
            pltpu.VMEM((8, 128), jnp.float32),
            pltpu.SMEM((1,), jnp.int32),
        ],
    )(*inputs)
    return out


def _row_spec(blk, ncol, nblk):
    return pl.BlockSpec((blk, ncol), lambda i: (jnp.minimum(i, nblk - 1), 0))


def _full_spec(shape):
    nd = len(shape)
    return pl.BlockSpec(shape, lambda i: (0,) * nd)


def _sc_gather(table, idx, chunk):
    nrow, ncol = idx.shape[0], table.shape[1]
    info = plsc.get_sparse_core_info()
    nw = info.num_cores * info.num_subcores
    per_w = nrow // nw
    assert per_w % chunk == 0 and chunk % 8 == 0

    mesh = plsc.VectorSubcoreMesh(core_axis_name="c", subcore_axis_name="s")

    @functools.partial(
        pl.kernel,
        mesh=mesh,
        out_type=jax.ShapeDtypeStruct((nrow, ncol), jnp.float32),
        scratch_types=[
            pltpu.VMEM((chunk,), jnp.int32),
            pltpu.VMEM((chunk, ncol), jnp.float32),
            pltpu.SemaphoreType.DMA,
        ],
    )
    def k(table_hbm, idx_hbm, out_hbm, idx_v, rows_v, sem):
        wid = lax.axis_index("s") * info.num_cores + lax.axis_index("c")
        base = wid * per_w

        def body(j, carry):
            off = base + j * chunk
            pltpu.sync_copy(idx_hbm.at[pl.ds(off, chunk)], idx_v)
            pltpu.async_copy(table_hbm.at[idx_v], rows_v, sem).wait()
            pltpu.sync_copy(rows_v, out_hbm.at[pl.ds(off, chunk)])
            return carry

        lax.fori_loop(0, per_w // chunk, body, 0)

    return k(table, idx)


def _seg_ends(ids, nseg, zero_row):
    seg = jnp.arange(nseg, dtype=ids.dtype)
    sl = jnp.searchsorted(ids, seg, side="left").astype(jnp.int32)
    sr = jnp.searchsorted(ids, seg, side="right").astype(jnp.int32)
    return jnp.where(sr > sl, sr - 1, zero_row)


def _pad_idx(idx, mult):
    pad = (-idx.shape[0]) % mult
    if pad:
        idx = jnp.concatenate([idx, jnp.zeros((pad,), idx.dtype)])
    return idx


def kernel(relatives, cluster, relatives2, indices2, cluster2, relatives3, cluster3,
           W1a, b1a, W1b, b1b, W1c, b1c,
           W2r1, b2r1, W2r2, b2r2, W2m1, b2m1, W2m2, b2m2,
           W3r1, b3r1, W3r2, b3r2, W3m1, b3m1, W3m2, b3m2):
    n1 = relatives.shape[0]
    m2 = relatives2.shape[0]
    m3 = relatives3.shape[0]
    c1 = 100000
    c2 = m3
    c3 = 10000

    blk = 4000
    nb1 = n1 // blk
    nb2 = m2 // blk
    blk3 = 5000
    nb3 = m3 // blk3

    b1a_, b1b_, b1c_ = (b.reshape(1, -1) for b in (b1a, b1b, b1c))
    b2r1_, b2r2_, b2m1_, b2m2_ = (b.reshape(1, -1) for b in (b2r1, b2r2, b2m1, b2m2))
    b3r1_, b3r2_, b3m1_, b3m2_ = (b.reshape(1, -1) for b in (b3r1, b3r2, b3m1, b3m2))

    ids1 = cluster.reshape(-1, 1).astype(jnp.int32)
    ids2 = cluster2.reshape(-1, 1).astype(jnp.int32)
    ids3 = cluster3.reshape(-1, 1).astype(jnp.int32)

    def stage1(x_ref, id_ref, wa, ba, wb, bb, wc, bc, out_ref, carry, cid):
        i = pl.program_id(0)

        @pl.when(i == 0)
        def _():
            cid[0] = -1
            carry[...] = jnp.full((8, 128), _NEG, jnp.float32)

        @pl.when(i == nb1)
        def _():
            out_ref[...] = jnp.zeros((blk, 128), jnp.float32)

        @pl.when(i < nb1)
        def _():
            x = x_ref[...]
            h = jnp.maximum(_affine(x, wa, ba), 0.0)
            h = jnp.maximum(_affine(h, wb, bb), 0.0)
            h = _affine(h, wc, bc)
            hp = jnp.concatenate([h, jnp.zeros((blk, 123), jnp.float32)], axis=1)
            out_ref[...] = _carry_fold(hp, id_ref[...], carry, cid, blk, 128)

    m1 = _stage_call(
        stage1, nb1, blk, 128,
        (relatives, ids1, W1a, b1a_, W1b, b1b_, W1c, b1c_),
        [_row_spec(blk, 3, nb1), _row_spec(blk, 1, nb1),
         _full_spec(W1a.shape), _full_spec(b1a_.shape),
         _full_spec(W1b.shape), _full_spec(b1b_.shape),
         _full_spec(W1c.shape), _full_spec(b1c_.shape)],
    )

    ends1 = _seg_ends(cluster.astype(jnp.int32), c1, jnp.int32(n1))
    gidx2 = ends1[indices2.astype(jnp.int32)]
    f1m = _sc_gather(m1, gidx2, 400)

    def stage2(x_ref, f_ref, id_ref, wr1, br1, wr2, br2, wm1, bm1, wm2, bm2,
               out_ref, carry, cid):
        i = pl.program_id(0)

        @pl.when(i == 0)
        def _():
            cid[0] = -1
            carry[...] = jnp.full((8, 128), _NEG, jnp.float32)

        @pl.when(i == nb2)
        def _():
            out_ref[...] = jnp.zeros((blk, 128), jnp.float32)

        @pl.when(i < nb2)
        def _():
            x = x_ref[...]
            r = jnp.maximum(_affine(x, wr1, br1), 0.0)
            r = jnp.maximum(_affine(r, wr2, br2), 0.0)
            comb = jnp.concatenate([r, f_ref[...][:, 0:5]], axis=1)
            h = jnp.maximum(_affine(comb, wm1, bm1), 0.0)
            h = _affine(h, wm2, bm2)
            cat = jnp.concatenate([h, r, jnp.zeros((blk, 98), jnp.float32)], axis=1)
            out_ref[...] = _carry_fold(cat, id_ref[...], carry, cid, blk, 128)

    m2rows = _stage_call(
        stage2, nb2, blk, 128,
        (relatives2, f1m, ids2, W2r1, b2r1_, W2r2, b2r2_, W2m1, b2m1_, W2m2, b2m2_),
        [_row_spec(blk, 3, nb2), _row_spec(blk, 128, nb2), _row_spec(blk, 1, nb2),
         _full_spec(W2r1.shape), _full_spec(b2r1_.shape),
         _full_spec(W2r2.shape), _full_spec(b2r2_.shape),
         _full_spec(W2m1.shape), _full_spec(b2m1_.shape),
         _full_spec(W2m2.shape), _full_spec(b2m2_.shape)],
    )

    ends2 = _pad_idx(_seg_ends(cluster2.astype(jnp.int32), c2, jnp.int32(m2)), 256)
    c2rows = _sc_gather(m2rows, ends2, 392)[:c2]

    def stage3(x_ref, c_ref, id_ref, wr1, br1, wr2, br2, wm1, bm1, wm2, bm2,
               out_ref, carry, cid):
        i = pl.program_id(0)

        @pl.when(i == 0)
        def _():
            cid[0] = -1
            carry[...] = jnp.full((8, 128), _NEG, jnp.float32)

        @pl.when(i == nb3)
        def _():
            out_ref[...] = jnp.zeros((blk3, 128), jnp.float32)

        @pl.when(i < nb3)
        def _():
            x = x_ref[...]
            r = jnp.maximum(_affine(x, wr1, br1), 0.0)
            r = jnp.maximum(_affine(r, wr2, br2), 0.0)
            comb = jnp.concatenate([r, c_ref[...][:, 0:30]], axis=1)
            h = jnp.maximum(_affine(comb, wm1, bm1), 0.0)
            h = _affine(h, wm2, bm2)
            cat = jnp.concatenate([h, r, jnp.zeros((blk3, 78), jnp.float32)], axis=1)
            out_ref[...] = _carry_fold(cat, id_ref[...], carry, cid, blk3, 128)

    m3rows = _stage_call(
        stage3, nb3, blk3, 128,
        (relatives3, c2rows, ids3, W3r1, b3r1_, W3r2, b3r2_, W3m1, b3m1_, W3m2, b3m2_),
        [_row_spec(blk3, 3, nb3), _row_spec(blk3, 128, nb3), _row_spec(blk3, 1, nb3),
         _full_spec(W3r1.shape), _full_spec(b3r1_.shape),
         _full_spec(W3r2.shape), _full_spec(b3r2_.shape),
         _full_spec(W3m1.shape), _full_spec(b3m1_.shape),
         _full_spec(W3m2.shape), _full_spec(b3m2_.shape)],
    )

    ends3 = _pad_idx(_seg_ends(cluster3.astype(jnp.int32), c3, jnp.int32(m3)), 256)
    out = _sc_gather(m3rows, ends3, ends3.shape[0] // 32)[:c3, :50]
    return out
